# baseline (speedup 1.0000x reference)
import functools

import jax
import jax.numpy as jnp
from jax import lax
from jax.experimental import pallas as pl
from jax.experimental.pallas import tpu as pltpu
from jax.experimental.pallas import tpu_sc as plsc


def _top3_body(xl_ref, xh_ref, fh_ref, idx_ref, w_ref, fht_ref):
    xl = xl_ref[0]
    xh = xh_ref[0]
    nh = xh.shape[0]
    lt = xl.shape[0]

    ql2 = jnp.sum(xl * xl, axis=1)[None, :]
    h2 = jnp.sum(xh * xh, axis=1, keepdims=True)
    cross = lax.dot_general(xh, xl, (((1,), (1,)), ((), ())),
                            preferred_element_type=jnp.float32)
    d2 = h2 + ql2 - 2.0 * cross

    big = jnp.float32(3.0e38)
    m0 = jnp.full((8, lt), big, jnp.float32)
    m1 = m0
    m2 = m0
    for s in range(nh // 8):
        v = d2[s * 8:(s + 1) * 8, :]
        h0 = jnp.maximum(m0, v)
        m0 = jnp.minimum(m0, v)
        h1 = jnp.maximum(m1, h0)
        m1 = jnp.minimum(m1, h0)
        m2 = jnp.minimum(m2, h1)

    cand = jnp.concatenate([m0, m1, m2], axis=0)
    nc = cand.shape[0]
    ci = lax.broadcasted_iota(jnp.int32, cand.shape, 0)
    d2s = []
    for _ in range(2):
        mk = jnp.min(cand, axis=0, keepdims=True)
        ik = jnp.min(jnp.where(cand == mk, ci, nc), axis=0, keepdims=True)
        cand = jnp.where(ci == ik, big, cand)
        d2s.append(mk)
    d2s.append(jnp.min(cand, axis=0, keepdims=True))

    iota = lax.broadcasted_iota(jnp.int32, d2.shape, 0)
    bi = pl.program_id(0)
    idxs = []
    ws = []
    for k in range(3):
        idxk = jnp.min(jnp.where(d2 == d2s[k], iota, nh), axis=0,
                       keepdims=True)
        idxs.append(idxk)
        ws.append(lax.rsqrt(jnp.maximum(d2s[k], 1e-12)))
    wsum = ws[0] + ws[1] + ws[2]
    rw = 1.0 / wsum
    idx_ref[0] = jnp.concatenate(idxs, axis=0) + bi * nh
    wexp = [jnp.broadcast_to(jnp.swapaxes(w * rw, 0, 1), (lt, 16))[None]
            for w in ws]
    w_ref[0] = jnp.concatenate(wexp, axis=0)

    @pl.when(pl.program_id(1) == 0)
    def _():
        fht_ref[0] = jnp.swapaxes(fh_ref[0], 0, 1)


def _make_sc_gather(P, Ch, nw, chunk):
    pt_per_w = P // nw
    nchunks = pt_per_w // chunk
    mesh = plsc.VectorSubcoreMesh(core_axis_name="c", subcore_axis_name="s")

    @functools.partial(
        pl.kernel,
        out_type=jax.ShapeDtypeStruct((P, Ch), jnp.float32),
        mesh=mesh,
        scratch_types=[
            pltpu.VMEM((chunk,), jnp.int32),
            pltpu.VMEM((chunk,), jnp.int32),
            pltpu.VMEM((chunk,), jnp.int32),
            pltpu.VMEM((chunk, 16), jnp.float32),
            pltpu.VMEM((chunk, 16), jnp.float32),
            pltpu.VMEM((chunk, 16), jnp.float32),
            pltpu.VMEM((chunk, Ch), jnp.float32),
            pltpu.VMEM((chunk, Ch), jnp.float32),
            pltpu.VMEM((chunk, Ch), jnp.float32),
            pltpu.VMEM((chunk, Ch), jnp.float32),
            pltpu.SemaphoreType.DMA,
        ],
    )
    def sc_gather(fht_hbm, idx_hbm, w_hbm, out_hbm,
                  i0, i1, i2, w0, w1, w2, r0, r1, r2, fi, sem):
        wid = lax.axis_index("s") * 2 + lax.axis_index("c")
        base = wid * pt_per_w
        for ciq in range(nchunks):
            start = base + ciq * chunk
            pltpu.sync_copy(idx_hbm.at[0, pl.ds(start, chunk)], i0)
            pltpu.sync_copy(idx_hbm.at[1, pl.ds(start, chunk)], i1)
            pltpu.sync_copy(idx_hbm.at[2, pl.ds(start, chunk)], i2)
            pltpu.sync_copy(w_hbm.at[0, pl.ds(start, chunk)], w0)
            pltpu.sync_copy(w_hbm.at[1, pl.ds(start, chunk)], w1)
            pltpu.sync_copy(w_hbm.at[2, pl.ds(start, chunk)], w2)
            g0 = pltpu.async_copy(fht_hbm.at[i0], r0, sem)
            g1 = pltpu.async_copy(fht_hbm.at[i1], r1, sem)
            g2 = pltpu.async_copy(fht_hbm.at[i2], r2, sem)
            g0.wait()
            g1.wait()
            g2.wait()

            def point_body(p, carry):
                wv0 = w0[p]
                wv1 = w1[p]
                wv2 = w2[p]
                for g in range(Ch // 16):
                    sl = pl.ds(g * 16, 16)
                    fi[p, sl] = (r0[p, sl] * wv0 + r1[p, sl] * wv1
                                 + r2[p, sl] * wv2)
                return carry

            lax.fori_loop(0, chunk, point_body, 0)
            pltpu.sync_copy(fi, out_hbm.at[pl.ds(start, chunk)])

    return sc_gather


def _linear_body(fi_ref, fl_ref, w_ref, b_ref, y_ref, stats_ref,
                 sum_acc, sq_acc, *, ch):
    fi = fi_ref[0].astype(jnp.bfloat16)
    fl = fl_ref[0].astype(jnp.bfloat16)
    w_hi = w_ref[:, :ch].astype(jnp.bfloat16)
    w_lo = w_ref[:, ch:].astype(jnp.bfloat16)
    y = (lax.dot_general(w_hi, fi, (((1,), (1,)), ((), ())),
                         preferred_element_type=jnp.float32)
         + lax.dot_general(w_lo, fl, (((1,), (0,)), ((), ())),
                           preferred_element_type=jnp.float32)
         + b_ref[...])
    y_ref[0] = y.astype(jnp.bfloat16)

    ys = jnp.sum(y, axis=1, keepdims=True)
    ysq = jnp.sum(y * y, axis=1, keepdims=True)
    step = pl.program_id(0) * pl.num_programs(1) + pl.program_id(1)

    @pl.when(step == 0)
    def _():
        sum_acc[...] = ys
        sq_acc[...] = ysq

    @pl.when(step > 0)
    def _():
        sum_acc[...] += ys
        sq_acc[...] += ysq

    @pl.when(step == pl.num_programs(0) * pl.num_programs(1) - 1)
    def _():
        stats_ref[...] = jnp.concatenate([sum_acc[...], sq_acc[...]], axis=1)


def _norm_body(y_ref, stats_ref, gamma_ref, beta_ref, o_ref, *, inv_n):
    mean = stats_ref[:, 0:1] * inv_n
    msq = stats_ref[:, 1:2] * inv_n
    var = msq - mean * mean
    rstd = lax.rsqrt(var + 1e-5)
    scale = gamma_ref[...] * rstd
    shift = beta_ref[...] - mean * scale
    o_ref[0] = jnp.maximum(y_ref[0].astype(jnp.float32) * scale + shift, 0.0)


def kernel(xyz_low, xyz_high, feat_low, feat_high, W, b, gamma, beta):
    B, Nl, _ = xyz_low.shape
    Nh = xyz_high.shape[1]
    Cl = feat_low.shape[1]
    Ch = feat_high.shape[1]
    out_ch = W.shape[0]
    P = B * Nl
    Lt = 2048
    nlt = Nl // Lt
    grid = (B, nlt)

    idx3, w3, fht = pl.pallas_call(
        _top3_body,
        grid=grid,
        in_specs=[
            pl.BlockSpec((1, Lt, 3), lambda bi, li: (bi, li, 0)),
            pl.BlockSpec((1, Nh, 3), lambda bi, li: (bi, 0, 0)),
            pl.BlockSpec((1, Ch, Nh), lambda bi, li: (bi, 0, 0)),
        ],
        out_specs=[
            pl.BlockSpec((1, 3, Lt), lambda bi, li: (bi * nlt + li, 0, 0)),
            pl.BlockSpec((1, 3, Lt, 16), lambda bi, li: (bi * nlt + li, 0, 0, 0)),
            pl.BlockSpec((1, Nh, Ch), lambda bi, li: (bi, 0, 0)),
        ],
        out_shape=[
            jax.ShapeDtypeStruct((B * nlt, 3, Lt), jnp.int32),
            jax.ShapeDtypeStruct((B * nlt, 3, Lt, 16), jnp.float32),
            jax.ShapeDtypeStruct((B, Nh, Ch), jnp.float32),
        ],
    )(xyz_low, xyz_high, feat_high)

    idx_flat = jnp.transpose(idx3, (1, 0, 2)).reshape(3, P)
    w_flat = jnp.transpose(w3, (1, 0, 2, 3)).reshape(3, P, 16)

    sc_gather = _make_sc_gather(P, Ch, 32, 64)
    fi = sc_gather(fht.reshape(B * Nh, Ch), idx_flat, w_flat)
    fi = fi.reshape(B, Nl, Ch)

    y, stats = pl.pallas_call(
        functools.partial(_linear_body, ch=Ch),
        grid=grid,
        in_specs=[
            pl.BlockSpec((1, Lt, Ch), lambda bi, li: (bi, li, 0)),
            pl.BlockSpec((1, Cl, Lt), lambda bi, li: (bi, 0, li)),
            pl.BlockSpec((out_ch, Cl + Ch), lambda bi, li: (0, 0)),
            pl.BlockSpec((out_ch, 1), lambda bi, li: (0, 0)),
        ],
        out_specs=[
            pl.BlockSpec((1, out_ch, Lt), lambda bi, li: (bi, 0, li)),
            pl.BlockSpec((out_ch, 2), lambda bi, li: (0, 0)),
        ],
        out_shape=[
            jax.ShapeDtypeStruct((B, out_ch, Nl), jnp.bfloat16),
            jax.ShapeDtypeStruct((out_ch, 2), jnp.float32),
        ],
        scratch_shapes=[
            pltpu.VMEM((out_ch, 1), jnp.float32),
            pltpu.VMEM((out_ch, 1), jnp.float32),
        ],
    )(fi, feat_low, W, b.reshape(out_ch, 1))

    Ln = 1024
    out = pl.pallas_call(
        functools.partial(_norm_body, inv_n=1.0 / (B * Nl)),
        grid=(B, Nl // Ln),
        in_specs=[
            pl.BlockSpec((1, out_ch, Ln), lambda bi, li: (bi, 0, li)),
            pl.BlockSpec((out_ch, 2), lambda bi, li: (0, 0)),
            pl.BlockSpec((out_ch, 1), lambda bi, li: (0, 0)),
            pl.BlockSpec((out_ch, 1), lambda bi, li: (0, 0)),
        ],
        out_specs=pl.BlockSpec((1, out_ch, Ln), lambda bi, li: (bi, 0, li)),
        out_shape=jax.ShapeDtypeStruct((B, out_ch, Nl), jnp.float32),
    )(y, stats, gamma.reshape(out_ch, 1), beta.reshape(out_ch, 1))
    return out


# speedup vs baseline: 1.1811x; 1.1811x over previous
import functools

import jax
import jax.numpy as jnp
from jax import lax
from jax.experimental import pallas as pl
from jax.experimental.pallas import tpu as pltpu
from jax.experimental.pallas import tpu_sc as plsc


def _top3_body(xl_ref, xh_ref, fh_ref, idx_ref, w_ref, fht_ref):
    xl = xl_ref[0]
    xh = xh_ref[0]
    nh = xh.shape[0]
    lt = xl.shape[0]

    ql2 = jnp.sum(xl * xl, axis=1)[None, :]
    h2 = jnp.sum(xh * xh, axis=1, keepdims=True)
    cross = lax.dot_general(xh, xl, (((1,), (1,)), ((), ())),
                            preferred_element_type=jnp.float32)
    d2 = h2 + ql2 - 2.0 * cross

    big = jnp.float32(3.0e38)
    m0 = jnp.full((8, lt), big, jnp.float32)
    m1 = m0
    m2 = m0
    for s in range(nh // 8):
        v = d2[s * 8:(s + 1) * 8, :]
        h0 = jnp.maximum(m0, v)
        m0 = jnp.minimum(m0, v)
        h1 = jnp.maximum(m1, h0)
        m1 = jnp.minimum(m1, h0)
        m2 = jnp.minimum(m2, h1)

    cand = jnp.concatenate([m0, m1, m2], axis=0)
    nc = cand.shape[0]
    ci = lax.broadcasted_iota(jnp.int32, cand.shape, 0)
    d2s = []
    for _ in range(2):
        mk = jnp.min(cand, axis=0, keepdims=True)
        ik = jnp.min(jnp.where(cand == mk, ci, nc), axis=0, keepdims=True)
        cand = jnp.where(ci == ik, big, cand)
        d2s.append(mk)
    d2s.append(jnp.min(cand, axis=0, keepdims=True))

    iota = lax.broadcasted_iota(jnp.int32, d2.shape, 0)
    bi = pl.program_id(0)
    idxs = []
    ws = []
    for k in range(3):
        idxk = jnp.min(jnp.where(d2 == d2s[k], iota, nh), axis=0,
                       keepdims=True)
        idxs.append(idxk)
        ws.append(lax.rsqrt(jnp.maximum(d2s[k], 1e-12)))
    wsum = ws[0] + ws[1] + ws[2]
    rw = 1.0 / wsum
    idx_ref[...] = jnp.concatenate(idxs, axis=0) + bi * nh
    wexp = [jnp.broadcast_to(
        jnp.swapaxes(w * rw, 0, 1), (lt, 16))[:, None]
            for w in ws]
    w_ref[...] = jnp.concatenate(wexp, axis=1)

    @pl.when(pl.program_id(1) == 0)
    def _():
        fht_ref[0] = jnp.swapaxes(fh_ref[0], 0, 1)


def _make_sc_gather(P, Ch, nw, chunk):
    pt_per_w = P // nw
    nchunks = pt_per_w // chunk
    mesh = plsc.VectorSubcoreMesh(core_axis_name="c", subcore_axis_name="s")

    @functools.partial(
        pl.kernel,
        out_type=jax.ShapeDtypeStruct((P, Ch), jnp.float32),
        mesh=mesh,
        scratch_types=[
            pltpu.VMEM((3, pt_per_w), jnp.int32),
            pltpu.VMEM((chunk, 3, 16), jnp.float32),
            pltpu.VMEM((chunk, 3, 16), jnp.float32),
            pltpu.VMEM((chunk, Ch), jnp.float32),
            pltpu.VMEM((chunk, Ch), jnp.float32),
            pltpu.VMEM((chunk, Ch), jnp.float32),
            pltpu.VMEM((chunk, Ch), jnp.float32),
            pltpu.VMEM((chunk, Ch), jnp.float32),
            pltpu.VMEM((chunk, Ch), jnp.float32),
            pltpu.VMEM((chunk, Ch), jnp.float32),
            pltpu.VMEM((chunk, Ch), jnp.float32),
            pltpu.SemaphoreType.DMA,
            pltpu.SemaphoreType.DMA,
            pltpu.SemaphoreType.DMA,
            pltpu.SemaphoreType.DMA,
        ],
    )
    def sc_gather(fht_hbm, idx_hbm, w_hbm, out_hbm,
                  ibuf, wb0, wb1, ra0, ra1, ra2, rb0, rb1, rb2,
                  fia, fib, sga, sgb, ssa, ssb):
        wid = lax.axis_index("s") * 2 + lax.axis_index("c")
        base = wid * pt_per_w
        pltpu.sync_copy(idx_hbm.at[:, pl.ds(base, pt_per_w)], ibuf)
        wbufs = (wb0, wb1)
        rbufs = ((ra0, ra1, ra2), (rb0, rb1, rb2))
        fibufs = (fia, fib)
        gsems = (sga, sgb)
        ssems = (ssa, ssb)
        gh = {}
        sh = {}

        def fire(slot, ciq):
            start = ciq * chunk
            pltpu.sync_copy(w_hbm.at[pl.ds(base + start, chunk)],
                            wbufs[slot])
            gh[slot] = [
                pltpu.async_copy(
                    fht_hbm.at[ibuf.at[k, pl.ds(start, chunk)]],
                    rbufs[slot][k], gsems[slot])
                for k in range(3)]

        fire(0, 0)
        for ciq in range(nchunks):
            slot = ciq % 2
            if ciq + 1 < nchunks:
                fire(1 - slot, ciq + 1)
            for h in gh[slot]:
                h.wait()
            if ciq >= 2:
                sh[slot].wait()
            r0, r1, r2 = rbufs[slot]
            wb = wbufs[slot]
            fi = fibufs[slot]

            def point_body(p, carry):
                wv0 = wb[p, 0]
                wv1 = wb[p, 1]
                wv2 = wb[p, 2]
                for g in range(Ch // 16):
                    sl = pl.ds(g * 16, 16)
                    fi[p, sl] = (r0[p, sl] * wv0 + r1[p, sl] * wv1
                                 + r2[p, sl] * wv2)
                return carry

            lax.fori_loop(0, chunk, point_body, 0)
            sh[slot] = pltpu.async_copy(
                fi, out_hbm.at[pl.ds(base + ciq * chunk, chunk)], ssems[slot])
        sh[0].wait()
        sh[1].wait()

    return sc_gather


def _linear_body(fi_ref, fl_ref, w_ref, b_ref, y_ref, stats_ref,
                 sum_acc, sq_acc, *, ch):
    fi = fi_ref[0].astype(jnp.bfloat16)
    fl = fl_ref[0].astype(jnp.bfloat16)
    w_hi = w_ref[:, :ch].astype(jnp.bfloat16)
    w_lo = w_ref[:, ch:].astype(jnp.bfloat16)
    y = (lax.dot_general(w_hi, fi, (((1,), (1,)), ((), ())),
                         preferred_element_type=jnp.float32)
         + lax.dot_general(w_lo, fl, (((1,), (0,)), ((), ())),
                           preferred_element_type=jnp.float32)
         + b_ref[...])
    y_ref[0] = y.astype(jnp.bfloat16)

    ys = jnp.sum(y, axis=1, keepdims=True)
    ysq = jnp.sum(y * y, axis=1, keepdims=True)
    step = pl.program_id(0) * pl.num_programs(1) + pl.program_id(1)

    @pl.when(step == 0)
    def _():
        sum_acc[...] = ys
        sq_acc[...] = ysq

    @pl.when(step > 0)
    def _():
        sum_acc[...] += ys
        sq_acc[...] += ysq

    @pl.when(step == pl.num_programs(0) * pl.num_programs(1) - 1)
    def _():
        stats_ref[...] = jnp.concatenate([sum_acc[...], sq_acc[...]], axis=1)


def _norm_body(y_ref, stats_ref, gamma_ref, beta_ref, o_ref, *, inv_n):
    mean = stats_ref[:, 0:1] * inv_n
    msq = stats_ref[:, 1:2] * inv_n
    var = msq - mean * mean
    rstd = lax.rsqrt(var + 1e-5)
    scale = gamma_ref[...] * rstd
    shift = beta_ref[...] - mean * scale
    o_ref[0] = jnp.maximum(y_ref[0].astype(jnp.float32) * scale + shift, 0.0)


def kernel(xyz_low, xyz_high, feat_low, feat_high, W, b, gamma, beta):
    B, Nl, _ = xyz_low.shape
    Nh = xyz_high.shape[1]
    Cl = feat_low.shape[1]
    Ch = feat_high.shape[1]
    out_ch = W.shape[0]
    P = B * Nl
    Lt = 2048
    nlt = Nl // Lt
    grid = (B, nlt)

    idx3, w3, fht = pl.pallas_call(
        _top3_body,
        grid=grid,
        in_specs=[
            pl.BlockSpec((1, Lt, 3), lambda bi, li: (bi, li, 0)),
            pl.BlockSpec((1, Nh, 3), lambda bi, li: (bi, 0, 0)),
            pl.BlockSpec((1, Ch, Nh), lambda bi, li: (bi, 0, 0)),
        ],
        out_specs=[
            pl.BlockSpec((3, Lt), lambda bi, li: (0, bi * nlt + li)),
            pl.BlockSpec((Lt, 3, 16), lambda bi, li: (bi * nlt + li, 0, 0)),
            pl.BlockSpec((1, Nh, Ch), lambda bi, li: (bi, 0, 0)),
        ],
        out_shape=[
            jax.ShapeDtypeStruct((3, P), jnp.int32),
            jax.ShapeDtypeStruct((P, 3, 16), jnp.float32),
            jax.ShapeDtypeStruct((B, Nh, Ch), jnp.float32),
        ],
    )(xyz_low, xyz_high, feat_high)

    sc_gather = _make_sc_gather(P, Ch, 32, 32)
    fi = sc_gather(fht.reshape(B * Nh, Ch), idx3, w3).reshape(B, Nl, Ch)

    y, stats = pl.pallas_call(
        functools.partial(_linear_body, ch=Ch),
        grid=grid,
        in_specs=[
            pl.BlockSpec((1, Lt, Ch), lambda bi, li: (bi, li, 0)),
            pl.BlockSpec((1, Cl, Lt), lambda bi, li: (bi, 0, li)),
            pl.BlockSpec((out_ch, Cl + Ch), lambda bi, li: (0, 0)),
            pl.BlockSpec((out_ch, 1), lambda bi, li: (0, 0)),
        ],
        out_specs=[
            pl.BlockSpec((1, out_ch, Lt), lambda bi, li: (bi, 0, li)),
            pl.BlockSpec((out_ch, 2), lambda bi, li: (0, 0)),
        ],
        out_shape=[
            jax.ShapeDtypeStruct((B, out_ch, Nl), jnp.bfloat16),
            jax.ShapeDtypeStruct((out_ch, 2), jnp.float32),
        ],
        scratch_shapes=[
            pltpu.VMEM((out_ch, 1), jnp.float32),
            pltpu.VMEM((out_ch, 1), jnp.float32),
        ],
    )(fi, feat_low, W, b.reshape(out_ch, 1))

    Ln = 1024
    out = pl.pallas_call(
        functools.partial(_norm_body, inv_n=1.0 / (B * Nl)),
        grid=(B, Nl // Ln),
        in_specs=[
            pl.BlockSpec((1, out_ch, Ln), lambda bi, li: (bi, 0, li)),
            pl.BlockSpec((out_ch, 2), lambda bi, li: (0, 0)),
            pl.BlockSpec((out_ch, 1), lambda bi, li: (0, 0)),
            pl.BlockSpec((out_ch, 1), lambda bi, li: (0, 0)),
        ],
        out_specs=pl.BlockSpec((1, out_ch, Ln), lambda bi, li: (bi, 0, li)),
        out_shape=jax.ShapeDtypeStruct((B, out_ch, Nl), jnp.float32),
    )(y, stats, gamma.reshape(out_ch, 1), beta.reshape(out_ch, 1))
    return out


# speedup vs baseline: 1.2059x; 1.0210x over previous
import functools

import jax
import jax.numpy as jnp
from jax import lax
from jax.experimental import pallas as pl
from jax.experimental.pallas import tpu as pltpu
from jax.experimental.pallas import tpu_sc as plsc


def _top3_body(xl_ref, xh_ref, fh_ref, idx_ref, w_ref, fht_ref):
    xl = xl_ref[0]
    xh = xh_ref[0]
    nh = xh.shape[0]
    lt = xl.shape[0]

    ql2 = jnp.sum(xl * xl, axis=1)[None, :]
    h2 = jnp.sum(xh * xh, axis=1, keepdims=True)
    cross = lax.dot_general(xh, xl, (((1,), (1,)), ((), ())),
                            preferred_element_type=jnp.float32)
    d2 = h2 + ql2 - 2.0 * cross

    big = jnp.float32(3.0e38)
    m0 = jnp.full((8, lt), big, jnp.float32)
    m1 = m0
    m2 = m0
    for s in range(nh // 8):
        v = d2[s * 8:(s + 1) * 8, :]
        h0 = jnp.maximum(m0, v)
        m0 = jnp.minimum(m0, v)
        h1 = jnp.maximum(m1, h0)
        m1 = jnp.minimum(m1, h0)
        m2 = jnp.minimum(m2, h1)

    cand = jnp.concatenate([m0, m1, m2], axis=0)
    nc = cand.shape[0]
    ci = lax.broadcasted_iota(jnp.int32, cand.shape, 0)
    d2s = []
    for _ in range(2):
        mk = jnp.min(cand, axis=0, keepdims=True)
        ik = jnp.min(jnp.where(cand == mk, ci, nc), axis=0, keepdims=True)
        cand = jnp.where(ci == ik, big, cand)
        d2s.append(mk)
    d2s.append(jnp.min(cand, axis=0, keepdims=True))

    iota = lax.broadcasted_iota(jnp.int32, d2.shape, 0)
    bi = pl.program_id(0)
    idxs = []
    ws = []
    for k in range(3):
        idxk = jnp.min(jnp.where(d2 == d2s[k], iota, nh), axis=0,
                       keepdims=True)
        idxs.append(idxk)
        ws.append(lax.rsqrt(jnp.maximum(d2s[k], 1e-12)))
    wsum = ws[0] + ws[1] + ws[2]
    rw = 1.0 / wsum
    idx_ref[...] = jnp.concatenate(idxs, axis=0) + bi * nh
    wexp = [jnp.broadcast_to(
        jnp.swapaxes(w * rw, 0, 1), (lt, 16))[:, None]
            for w in ws]
    w_ref[...] = jnp.concatenate(wexp, axis=1)

    @pl.when(pl.program_id(1) == 0)
    def _():
        fht_ref[0] = jnp.swapaxes(fh_ref[0], 0, 1)


def _make_sc_gather(P, Ch, nw, chunk):
    pt_per_w = P // nw
    nchunks = pt_per_w // chunk
    mesh = plsc.VectorSubcoreMesh(core_axis_name="c", subcore_axis_name="s")

    @functools.partial(
        pl.kernel,
        out_type=jax.ShapeDtypeStruct((P, Ch), jnp.float32),
        mesh=mesh,
        scratch_types=[
            pltpu.VMEM((3, pt_per_w), jnp.int32),
            pltpu.VMEM((chunk, 3, 16), jnp.float32),
            pltpu.VMEM((chunk, 3, 16), jnp.float32),
            pltpu.VMEM((chunk, Ch), jnp.float32),
            pltpu.VMEM((chunk, Ch), jnp.float32),
            pltpu.VMEM((chunk, Ch), jnp.float32),
            pltpu.VMEM((chunk, Ch), jnp.float32),
            pltpu.VMEM((chunk, Ch), jnp.float32),
            pltpu.VMEM((chunk, Ch), jnp.float32),
            pltpu.VMEM((chunk, Ch), jnp.float32),
            pltpu.VMEM((chunk, Ch), jnp.float32),
            pltpu.SemaphoreType.DMA,
            pltpu.SemaphoreType.DMA,
            pltpu.SemaphoreType.DMA,
            pltpu.SemaphoreType.DMA,
        ],
    )
    def sc_gather(fht_hbm, idx_hbm, w_hbm, out_hbm,
                  ibuf, wb0, wb1, ra0, ra1, ra2, rb0, rb1, rb2,
                  fia, fib, sga, sgb, ssa, ssb):
        wid = lax.axis_index("s") * 2 + lax.axis_index("c")
        base = wid * pt_per_w
        pltpu.sync_copy(idx_hbm.at[:, pl.ds(base, pt_per_w)], ibuf)
        wbufs = (wb0, wb1)
        rbufs = ((ra0, ra1, ra2), (rb0, rb1, rb2))
        fibufs = (fia, fib)
        gsems = (sga, sgb)
        ssems = (ssa, ssb)
        gh = {}
        sh = {}

        def fire(slot, ciq):
            start = ciq * chunk
            pltpu.sync_copy(w_hbm.at[pl.ds(base + start, chunk)],
                            wbufs[slot])
            gh[slot] = [
                pltpu.async_copy(
                    fht_hbm.at[ibuf.at[k, pl.ds(start, chunk)]],
                    rbufs[slot][k], gsems[slot])
                for k in range(3)]

        fire(0, 0)
        for ciq in range(nchunks):
            slot = ciq % 2
            if ciq + 1 < nchunks:
                fire(1 - slot, ciq + 1)
            for h in gh[slot]:
                h.wait()
            if ciq >= 2:
                sh[slot].wait()
            r0, r1, r2 = rbufs[slot]
            wb = wbufs[slot]
            fi = fibufs[slot]

            def point_body(p, carry):
                wv0 = wb[p, 0]
                wv1 = wb[p, 1]
                wv2 = wb[p, 2]
                for g in range(Ch // 16):
                    sl = pl.ds(g * 16, 16)
                    fi[p, sl] = (r0[p, sl] * wv0 + r1[p, sl] * wv1
                                 + r2[p, sl] * wv2)
                return carry

            lax.fori_loop(0, chunk, point_body, 0)
            sh[slot] = pltpu.async_copy(
                fi, out_hbm.at[pl.ds(base + ciq * chunk, chunk)], ssems[slot])
        sh[0].wait()
        sh[1].wait()

    return sc_gather


def _linear_body(fi_ref, fl_ref, w_ref, b_ref, y_ref, stats_ref,
                 sum_acc, sq_acc, *, ch):
    fi = fi_ref[0].astype(jnp.bfloat16)
    fl = fl_ref[0].astype(jnp.bfloat16)
    w_hi = w_ref[:, :ch].astype(jnp.bfloat16)
    w_lo = w_ref[:, ch:].astype(jnp.bfloat16)
    y = (lax.dot_general(w_hi, fi, (((1,), (1,)), ((), ())),
                         preferred_element_type=jnp.float32)
         + lax.dot_general(w_lo, fl, (((1,), (0,)), ((), ())),
                           preferred_element_type=jnp.float32)
         + b_ref[...])
    y_ref[0] = y.astype(jnp.bfloat16)

    ys = jnp.sum(y, axis=1, keepdims=True)
    ysq = jnp.sum(y * y, axis=1, keepdims=True)
    step = pl.program_id(0) * pl.num_programs(1) + pl.program_id(1)

    @pl.when(step == 0)
    def _():
        sum_acc[...] = ys
        sq_acc[...] = ysq

    @pl.when(step > 0)
    def _():
        sum_acc[...] += ys
        sq_acc[...] += ysq

    @pl.when(step == pl.num_programs(0) * pl.num_programs(1) - 1)
    def _():
        stats_ref[...] = jnp.concatenate([sum_acc[...], sq_acc[...]], axis=1)


def _norm_body(y_ref, s0_ref, s1_ref, gamma_ref, beta_ref, o_ref, *, inv_n):
    stats = s0_ref[...] + s1_ref[...]
    mean = stats[:, 0:1] * inv_n
    msq = stats[:, 1:2] * inv_n
    var = msq - mean * mean
    rstd = lax.rsqrt(var + 1e-5)
    scale = gamma_ref[...] * rstd
    shift = beta_ref[...] - mean * scale
    o_ref[0] = jnp.maximum(y_ref[0].astype(jnp.float32) * scale + shift, 0.0)


def kernel(xyz_low, xyz_high, feat_low, feat_high, W, b, gamma, beta):
    B, Nl, _ = xyz_low.shape
    Nh = xyz_high.shape[1]
    Cl = feat_low.shape[1]
    Ch = feat_high.shape[1]
    out_ch = W.shape[0]
    Lt = 2048
    nlt = Nl // Lt
    Bh = B // 2
    Ph = Bh * Nl
    grid_h = (Bh, nlt)
    b_col = b.reshape(out_ch, 1)
    sc_gather = _make_sc_gather(Ph, Ch, 32, 32)

    def half(xl_h, xh_h, fl_h, fh_h):
        idx3, w3, fht = pl.pallas_call(
            _top3_body,
            grid=grid_h,
            in_specs=[
                pl.BlockSpec((1, Lt, 3), lambda bi, li: (bi, li, 0)),
                pl.BlockSpec((1, Nh, 3), lambda bi, li: (bi, 0, 0)),
                pl.BlockSpec((1, Ch, Nh), lambda bi, li: (bi, 0, 0)),
            ],
            out_specs=[
                pl.BlockSpec((3, Lt), lambda bi, li: (0, bi * nlt + li)),
                pl.BlockSpec((Lt, 3, 16), lambda bi, li: (bi * nlt + li, 0, 0)),
                pl.BlockSpec((1, Nh, Ch), lambda bi, li: (bi, 0, 0)),
            ],
            out_shape=[
                jax.ShapeDtypeStruct((3, Ph), jnp.int32),
                jax.ShapeDtypeStruct((Ph, 3, 16), jnp.float32),
                jax.ShapeDtypeStruct((Bh, Nh, Ch), jnp.float32),
            ],
        )(xl_h, xh_h, fh_h)

        fi = sc_gather(fht.reshape(Bh * Nh, Ch), idx3, w3)
        fi = fi.reshape(Bh, Nl, Ch)

        y, stats = pl.pallas_call(
            functools.partial(_linear_body, ch=Ch),
            grid=grid_h,
            in_specs=[
                pl.BlockSpec((1, Lt, Ch), lambda bi, li: (bi, li, 0)),
                pl.BlockSpec((1, Cl, Lt), lambda bi, li: (bi, 0, li)),
                pl.BlockSpec((out_ch, Cl + Ch), lambda bi, li: (0, 0)),
                pl.BlockSpec((out_ch, 1), lambda bi, li: (0, 0)),
            ],
            out_specs=[
                pl.BlockSpec((1, out_ch, Lt), lambda bi, li: (bi, 0, li)),
                pl.BlockSpec((out_ch, 2), lambda bi, li: (0, 0)),
            ],
            out_shape=[
                jax.ShapeDtypeStruct((Bh, out_ch, Nl), jnp.bfloat16),
                jax.ShapeDtypeStruct((out_ch, 2), jnp.float32),
            ],
            scratch_shapes=[
                pltpu.VMEM((out_ch, 1), jnp.float32),
                pltpu.VMEM((out_ch, 1), jnp.float32),
            ],
        )(fi, fl_h, W, b_col)
        return y, stats

    y0, s0 = half(xyz_low[:Bh], xyz_high[:Bh], feat_low[:Bh], feat_high[:Bh])
    y1, s1 = half(xyz_low[Bh:], xyz_high[Bh:], feat_low[Bh:], feat_high[Bh:])

    Ln = 1024
    gamma_col = gamma.reshape(out_ch, 1)
    beta_col = beta.reshape(out_ch, 1)

    def norm_half(y_h):
        return pl.pallas_call(
            functools.partial(_norm_body, inv_n=1.0 / (B * Nl)),
            grid=(Bh, Nl // Ln),
            in_specs=[
                pl.BlockSpec((1, out_ch, Ln), lambda bi, li: (bi, 0, li)),
                pl.BlockSpec((out_ch, 2), lambda bi, li: (0, 0)),
                pl.BlockSpec((out_ch, 2), lambda bi, li: (0, 0)),
                pl.BlockSpec((out_ch, 1), lambda bi, li: (0, 0)),
                pl.BlockSpec((out_ch, 1), lambda bi, li: (0, 0)),
            ],
            out_specs=pl.BlockSpec((1, out_ch, Ln), lambda bi, li: (bi, 0, li)),
            out_shape=jax.ShapeDtypeStruct((Bh, out_ch, Nl), jnp.float32),
        )(y_h, s0, s1, gamma_col, beta_col)

    return jnp.concatenate([norm_half(y0), norm_half(y1)], axis=0)
